# baseline (speedup 1.0000x reference)
import functools

import jax
import jax.numpy as jnp
from jax import lax
from jax.experimental import pallas as pl
from jax.experimental.pallas import tpu as pltpu
from jax.experimental.pallas import tpu_sc as plsc

N_BITS = 20
OUT_DIM = 16
BATCH = 16384
NUM_CORES = 2
NUM_SUBCORES = 16
NW = NUM_CORES * NUM_SUBCORES
B_PER_W = BATCH // NW
SUB = 128
N_SUB = B_PER_W // SUB
LANES = 16
SUP_W = 128


def _body(img_hbm, enc_hbm, out_hbm, img_v, idx_v, sup_v, scale_v, rows_v,
          out_v, sem):
    wid = lax.axis_index("s") * NUM_CORES + lax.axis_index("c")
    base_row = wid * B_PER_W
    pltpu.sync_copy(img_hbm.at[pl.ds(base_row * N_BITS, B_PER_W * N_BITS)], img_v)

    lanes = lax.iota(jnp.int32, LANES)

    def sub_block(s, _):
        sub_base = s * SUB

        def index_chunk(cix, _):
            row0 = sub_base + cix * LANES
            flat0 = (row0 + lanes) * N_BITS
            idx = jnp.zeros((LANES,), jnp.int32)
            acc = jnp.zeros((LANES,), jnp.float32)
            for j in range(N_BITS):
                g = plsc.load_gather(img_v, [flat0 + j])
                bit = jnp.full((LANES,), 1 << j, jnp.int32)
                idx = idx + jnp.where(g > 0, bit, jnp.zeros((LANES,), jnp.int32))
                acc = acc + jnp.abs(g)
            idx_v[pl.ds(cix * LANES, LANES)] = idx
            sup_v[pl.ds(cix * LANES, LANES)] = idx >> 3
            scale_v[pl.ds(cix * LANES, LANES)] = acc * (1.0 / N_BITS)
            return _

        lax.fori_loop(0, SUB // LANES, index_chunk, 0)

        pltpu.async_copy(enc_hbm.at[sup_v], rows_v, sem).wait()

        def scale_chunk(cix, _):
            rbase = cix * LANES
            svec = scale_v[pl.ds(rbase, LANES)]
            offs = (idx_v[pl.ds(rbase, LANES)] & 7) << 4
            for k in range(LANES):
                off_b = jnp.broadcast_to(offs[k], (LANES,))
                s_b = jnp.broadcast_to(svec[k], (LANES,))
                row = jnp.full((LANES,), rbase + k, jnp.int32)
                vals = plsc.load_gather(rows_v, [row, off_b + lanes])
                out_v[pl.ds((rbase + k) * OUT_DIM, OUT_DIM)] = vals * s_b
            return _

        lax.fori_loop(0, SUB // LANES, scale_chunk, 0)

        pltpu.sync_copy(
            out_v, out_hbm.at[pl.ds((base_row + sub_base) * OUT_DIM,
                                    SUB * OUT_DIM)])
        return _

    lax.fori_loop(0, N_SUB, sub_block, 0)


@jax.jit
def kernel(image, encoding):
    mesh = plsc.VectorSubcoreMesh(
        core_axis_name="c", subcore_axis_name="s",
        num_cores=NUM_CORES, num_subcores=NUM_SUBCORES)
    k = functools.partial(
        pl.kernel,
        out_type=jax.ShapeDtypeStruct((BATCH * OUT_DIM,), jnp.float32),
        mesh=mesh,
        scratch_types=[
            pltpu.VMEM((B_PER_W * N_BITS,), jnp.float32),
            pltpu.VMEM((SUB,), jnp.int32),
            pltpu.VMEM((SUB,), jnp.int32),
            pltpu.VMEM((SUB,), jnp.float32),
            pltpu.VMEM((SUB, SUP_W), jnp.float32),
            pltpu.VMEM((SUB * OUT_DIM,), jnp.float32),
            pltpu.SemaphoreType.DMA,
        ],
        compiler_params=pltpu.CompilerParams(needs_layout_passes=False),
    )(_body)
    enc_wide = encoding.reshape(2 ** N_BITS // 8, SUP_W)
    out = k(image.reshape(-1), enc_wide)
    return out.reshape(BATCH, OUT_DIM)


# speedup vs baseline: 4.9760x; 4.9760x over previous
import functools

import jax
import jax.numpy as jnp
from jax import lax
from jax.experimental import pallas as pl
from jax.experimental.pallas import tpu as pltpu
from jax.experimental.pallas import tpu_sc as plsc

N_BITS = 20
OUT_DIM = 16
BATCH = 16384
NUM_CORES = 2
NUM_SUBCORES = 16
NW = NUM_CORES * NUM_SUBCORES
B_PER_W = BATCH // NW
NBLK = B_PER_W // 128
LANES = 16
HALF = 8 * (2 ** N_BITS)


def _body(img_hbm, enc_hbm, out_hbm, img_v, idx_v, scale_v, addr_v, gat_v, sem):
    wid = lax.axis_index("s") * NUM_CORES + lax.axis_index("c")
    base_row = wid * B_PER_W
    pltpu.sync_copy(img_hbm.at[pl.ds(base_row * N_BITS, B_PER_W * N_BITS)], img_v)

    lanes = lax.iota(jnp.int32, LANES)

    def index_chunk(cix, _):
        flat0 = (cix * LANES + lanes) * N_BITS
        idx = jnp.zeros((LANES,), jnp.int32)
        acc = jnp.zeros((LANES,), jnp.float32)
        for j in range(N_BITS):
            g = plsc.load_gather(img_v, [flat0 + j])
            bit = jnp.full((LANES,), 1 << j, jnp.int32)
            idx = idx + jnp.where(g > 0, bit, jnp.zeros((LANES,), jnp.int32))
            acc = acc + jnp.abs(g)
        idx_v[pl.ds(cix * LANES, LANES)] = idx
        scale_v[pl.ds(cix * LANES, LANES)] = acc * (1.0 / N_BITS)
        return _

    lax.fori_loop(0, B_PER_W // LANES, index_chunk, 0)

    def addr_chunk(cix, _):
        idx = idx_v[pl.ds(cix * LANES, LANES)]
        ebase = ((idx >> 7) << 10) + (idx & 127)
        for h in range(2):
            for cc in range(8):
                a = ebase + (h * HALF + cc * 128)
                dst = ((h * NBLK + (cix // 8)) * 8 + cc) * 128 + (cix % 8) * LANES
                addr_v[pl.ds(dst, LANES)] = a
        return _

    lax.fori_loop(0, B_PER_W // LANES, addr_chunk, 0)

    def gather_chunk(g, _):
        pltpu.async_copy(
            enc_hbm.at[addr_v.at[pl.ds(g * 128, 128)]],
            gat_v.at[pl.ds(g * 128, 128)], sem).wait()
        return _

    lax.fori_loop(0, 2 * NBLK * 8, gather_chunk, 0)

    def scale_chunk(cix, _):
        h_b2_cc, rm16 = cix // 8, cix % 8
        b2 = (h_b2_cc // 8) % NBLK
        s = scale_v[pl.ds(b2 * 128 + rm16 * LANES, LANES)]
        v = gat_v[pl.ds(cix * LANES, LANES)]
        gat_v[pl.ds(cix * LANES, LANES)] = v * s
        return _

    lax.fori_loop(0, 2 * NBLK * 8 * 8, scale_chunk, 0)

    half_w = NBLK * 8 * 128
    for h in range(2):
        pltpu.sync_copy(
            gat_v.at[pl.ds(h * half_w, half_w)],
            out_hbm.at[pl.ds(h * (BATCH // 128) * 1024 + wid * half_w, half_w)])


@jax.jit
def kernel(image, encoding):
    mesh = plsc.VectorSubcoreMesh(
        core_axis_name="c", subcore_axis_name="s",
        num_cores=NUM_CORES, num_subcores=NUM_SUBCORES)
    k = functools.partial(
        pl.kernel,
        out_type=jax.ShapeDtypeStruct((2 * (BATCH // 128) * 1024,), jnp.float32),
        mesh=mesh,
        scratch_types=[
            pltpu.VMEM((B_PER_W * N_BITS,), jnp.float32),
            pltpu.VMEM((B_PER_W,), jnp.int32),
            pltpu.VMEM((B_PER_W,), jnp.float32),
            pltpu.VMEM((B_PER_W * OUT_DIM,), jnp.int32),
            pltpu.VMEM((B_PER_W * OUT_DIM,), jnp.float32),
            pltpu.SemaphoreType.DMA,
        ],
        compiler_params=pltpu.CompilerParams(
            needs_layout_passes=False, use_tc_tiling_on_sc=False),
    )(_body)
    enc_flat = (encoding.reshape(8192, 128, 2, 8)
                .transpose(2, 0, 3, 1).reshape(-1))
    out = k(image.reshape(-1), enc_flat)
    return (out.reshape(2, BATCH // 128, 8, 128)
            .transpose(1, 3, 0, 2).reshape(BATCH, OUT_DIM))


# speedup vs baseline: 9.0551x; 1.8198x over previous
import functools

import jax
import jax.numpy as jnp
from jax import lax
from jax.experimental import pallas as pl
from jax.experimental.pallas import tpu as pltpu
from jax.experimental.pallas import tpu_sc as plsc

N_BITS = 20
OUT_DIM = 16
BATCH = 16384
NUM_CORES = 2
NUM_SUBCORES = 16
NW = NUM_CORES * NUM_SUBCORES
B_PER_W = BATCH // NW
NBLK = B_PER_W // 128
LANES = 16
HALF = 8 * (2 ** N_BITS)


def _body(img_hbm, enc_hbm, out_hbm, img_v, scale_v, addr_v, gat_v, sem):
    wid = lax.axis_index("s") * NUM_CORES + lax.axis_index("c")
    base_row = wid * B_PER_W
    pltpu.sync_copy(img_hbm.at[pl.ds(base_row * N_BITS, B_PER_W * N_BITS)], img_v)

    lanes = lax.iota(jnp.int32, LANES)

    def row_block(b2, _):
        def index_chunk(c8, _):
            cix = b2 * 8 + c8
            flat0 = (cix * LANES + lanes) * N_BITS
            idx = jnp.zeros((LANES,), jnp.int32)
            acc = jnp.zeros((LANES,), jnp.float32)
            for j in range(N_BITS):
                g = plsc.load_gather(img_v, [flat0 + j])
                bit = jnp.full((LANES,), 1 << j, jnp.int32)
                idx = idx + jnp.where(g > 0, bit,
                                      jnp.zeros((LANES,), jnp.int32))
                acc = acc + jnp.abs(g)
            scale_v[pl.ds(cix * LANES, LANES)] = acc * (1.0 / N_BITS)
            ebase = ((idx >> 7) << 10) + (idx & 127)
            for h in range(2):
                for cc in range(8):
                    a = ebase + (h * HALF + cc * 128)
                    dst = (((h * NBLK + b2) * 8 + cc) * 8 + c8) * LANES
                    addr_v[pl.ds(dst, LANES)] = a
            return _

        lax.fori_loop(0, 8, index_chunk, 0)

        for h in range(2):
            for cc in range(8):
                g = (h * NBLK + b2) * 8 + cc
                pltpu.async_copy(
                    enc_hbm.at[addr_v.at[pl.ds(g * 128, 128)]],
                    gat_v.at[pl.ds(g * 128, 128)], sem)
        return _

    lax.fori_loop(0, NBLK, row_block, 0)

    def gather_drain(g, _):
        pltpu.make_async_copy(
            enc_hbm.at[addr_v.at[pl.ds(g * 128, 128)]],
            gat_v.at[pl.ds(g * 128, 128)], sem).wait()
        return _

    lax.fori_loop(0, 2 * NBLK * 8, gather_drain, 0)

    def scale_chunk(cix, _):
        h_b2_cc, rm16 = cix // 8, cix % 8
        b2 = (h_b2_cc // 8) % NBLK
        s = scale_v[pl.ds(b2 * 128 + rm16 * LANES, LANES)]
        v = gat_v[pl.ds(cix * LANES, LANES)]
        gat_v[pl.ds(cix * LANES, LANES)] = v * s
        return _

    lax.fori_loop(0, 2 * NBLK * 8 * 8, scale_chunk, 0)

    half_w = NBLK * 8 * 128
    for h in range(2):
        pltpu.sync_copy(
            gat_v.at[pl.ds(h * half_w, half_w)],
            out_hbm.at[pl.ds(h * (BATCH // 128) * 1024 + wid * half_w, half_w)])


@jax.jit
def kernel(image, encoding):
    mesh = plsc.VectorSubcoreMesh(
        core_axis_name="c", subcore_axis_name="s",
        num_cores=NUM_CORES, num_subcores=NUM_SUBCORES)
    k = functools.partial(
        pl.kernel,
        out_type=jax.ShapeDtypeStruct((2 * (BATCH // 128) * 1024,), jnp.float32),
        mesh=mesh,
        scratch_types=[
            pltpu.VMEM((B_PER_W * N_BITS,), jnp.float32),
            pltpu.VMEM((B_PER_W,), jnp.float32),
            pltpu.VMEM((B_PER_W * OUT_DIM,), jnp.int32),
            pltpu.VMEM((B_PER_W * OUT_DIM,), jnp.float32),
            pltpu.SemaphoreType.DMA,
        ],
        compiler_params=pltpu.CompilerParams(
            needs_layout_passes=False, use_tc_tiling_on_sc=False),
    )(_body)
    enc_flat = (encoding.reshape(8192, 128, 2, 8)
                .transpose(2, 0, 3, 1).reshape(-1))
    out = k(image.reshape(-1), enc_flat)
    return (out.reshape(2, BATCH // 128, 8, 128)
            .transpose(1, 3, 0, 2).reshape(BATCH, OUT_DIM))


# speedup vs baseline: 9.1101x; 1.0061x over previous
import functools

import jax
import jax.numpy as jnp
from jax import lax
from jax.experimental import pallas as pl
from jax.experimental.pallas import tpu as pltpu
from jax.experimental.pallas import tpu_sc as plsc

N_BITS = 20
OUT_DIM = 16
BATCH = 16384
NUM_CORES = 2
NUM_SUBCORES = 16
NW = NUM_CORES * NUM_SUBCORES
B_PER_W = BATCH // NW
NBLK = B_PER_W // 128
LANES = 16
HALF = 8 * (2 ** N_BITS)


def _body(img_hbm, enc_hbm, out_hbm, img_v, scale_v, addr_v, gat_v, sem):
    wid = lax.axis_index("s") * NUM_CORES + lax.axis_index("c")
    base_row = wid * B_PER_W
    pltpu.sync_copy(img_hbm.at[pl.ds(base_row * N_BITS, B_PER_W * N_BITS)], img_v)

    lanes = lax.iota(jnp.int32, LANES)

    def row_block(b2, _):
        def index_chunk(c8, _):
            cix = b2 * 8 + c8
            flat0 = (cix * LANES + lanes) * N_BITS
            idx = jnp.zeros((LANES,), jnp.int32)
            acc = jnp.zeros((LANES,), jnp.float32)
            for j in range(N_BITS):
                g = plsc.load_gather(img_v, [flat0 + j])
                bit = jnp.full((LANES,), 1 << j, jnp.int32)
                idx = idx + jnp.where(g > 0, bit,
                                      jnp.zeros((LANES,), jnp.int32))
                acc = acc + jnp.abs(g)
            scale_v[pl.ds(cix * LANES, LANES)] = acc * (1.0 / N_BITS)
            ebase = ((idx >> 7) << 10) + (idx & 127)
            for h in range(2):
                for cc in range(8):
                    a = ebase + (h * HALF + cc * 128)
                    dst = (((h * NBLK + b2) * 8 + cc) * 8 + c8) * LANES
                    addr_v[pl.ds(dst, LANES)] = a
            return _

        lax.fori_loop(0, 8, index_chunk, 0)

        for h in range(2):
            for cc in range(8):
                g = (h * NBLK + b2) * 8 + cc
                pltpu.async_copy(
                    enc_hbm.at[addr_v.at[pl.ds(g * 128, 128)]],
                    gat_v.at[pl.ds(g * 128, 128)], sem)
        return _

    lax.fori_loop(0, NBLK, row_block, 0)

    def gather_drain(g, _):
        pltpu.make_async_copy(
            enc_hbm.at[addr_v.at[pl.ds(g * 128, 128)]],
            gat_v.at[pl.ds(g * 128, 128)], sem).wait()
        return _

    lax.fori_loop(0, 2 * NBLK * 8, gather_drain, 0)

    def scale_group(g, _):
        b2 = (g // 8) % NBLK
        for r8 in range(8):
            s = scale_v[pl.ds(b2 * 128 + r8 * LANES, LANES)]
            v = gat_v[pl.ds(g * 128 + r8 * LANES, LANES)]
            gat_v[pl.ds(g * 128 + r8 * LANES, LANES)] = v * s
        return _

    lax.fori_loop(0, 2 * NBLK * 8, scale_group, 0)

    half_w = NBLK * 8 * 128
    for h in range(2):
        pltpu.sync_copy(
            gat_v.at[pl.ds(h * half_w, half_w)],
            out_hbm.at[pl.ds(h * (BATCH // 128) * 1024 + wid * half_w, half_w)])


@jax.jit
def kernel(image, encoding):
    mesh = plsc.VectorSubcoreMesh(
        core_axis_name="c", subcore_axis_name="s",
        num_cores=NUM_CORES, num_subcores=NUM_SUBCORES)
    k = functools.partial(
        pl.kernel,
        out_type=jax.ShapeDtypeStruct((2 * (BATCH // 128) * 1024,), jnp.float32),
        mesh=mesh,
        scratch_types=[
            pltpu.VMEM((B_PER_W * N_BITS,), jnp.float32),
            pltpu.VMEM((B_PER_W,), jnp.float32),
            pltpu.VMEM((B_PER_W * OUT_DIM,), jnp.int32),
            pltpu.VMEM((B_PER_W * OUT_DIM,), jnp.float32),
            pltpu.SemaphoreType.DMA,
        ],
        compiler_params=pltpu.CompilerParams(
            needs_layout_passes=False, use_tc_tiling_on_sc=False),
    )(_body)
    enc_flat = (encoding.reshape(8192, 128, 2, 8)
                .transpose(2, 0, 3, 1).reshape(-1))
    out = k(image.reshape(-1), enc_flat)
    return (out.reshape(2, BATCH // 128, 8, 128)
            .transpose(1, 3, 0, 2).reshape(BATCH, OUT_DIM))


# speedup vs baseline: 12.5938x; 1.3824x over previous
import functools

import jax
import jax.numpy as jnp
from jax import lax
from jax.experimental import pallas as pl
from jax.experimental.pallas import tpu as pltpu
from jax.experimental.pallas import tpu_sc as plsc

N_BITS = 20
OUT_DIM = 16
BATCH = 16384
NUM_CORES = 2
NUM_SUBCORES = 16
NW = NUM_CORES * NUM_SUBCORES
B_PER_W = BATCH // NW
NBLK = B_PER_W // 128
LANES = 16
HALF = 8 * (2 ** N_BITS)


def _body(img_hbm, enc_hbm, out_hbm, img_v, scale_v, addr_v, gat_v, sem, sem2):
    wid = lax.axis_index("s") * NUM_CORES + lax.axis_index("c")
    base_row = wid * B_PER_W
    for j in range(N_BITS):
        pltpu.async_copy(
            img_hbm.at[pl.ds(j * BATCH + base_row, B_PER_W)],
            img_v.at[pl.ds(j * B_PER_W, B_PER_W)], sem2)
    for j in range(N_BITS):
        pltpu.make_async_copy(
            img_hbm.at[pl.ds(j * BATCH + base_row, B_PER_W)],
            img_v.at[pl.ds(j * B_PER_W, B_PER_W)], sem2).wait()

    lanes = lax.iota(jnp.int32, LANES)

    def row_block(b2, _):
        def index_chunk(c8, _):
            cix = b2 * 8 + c8
            idx = jnp.zeros((LANES,), jnp.int32)
            acc = jnp.zeros((LANES,), jnp.float32)
            for j in range(N_BITS):
                g = img_v[pl.ds(j * B_PER_W + cix * LANES, LANES)]
                bit = jnp.full((LANES,), 1 << j, jnp.int32)
                idx = idx + jnp.where(g > 0, bit,
                                      jnp.zeros((LANES,), jnp.int32))
                acc = acc + jnp.abs(g)
            scale_v[pl.ds(cix * LANES, LANES)] = acc * (1.0 / N_BITS)
            ebase = ((idx >> 7) << 10) + (idx & 127)
            for h in range(2):
                for cc in range(8):
                    a = ebase + (h * HALF + cc * 128)
                    dst = (((h * NBLK + b2) * 8 + cc) * 8 + c8) * LANES
                    addr_v[pl.ds(dst, LANES)] = a
            return _

        lax.fori_loop(0, 8, index_chunk, 0)

        for h in range(2):
            for cc in range(8):
                g = (h * NBLK + b2) * 8 + cc
                pltpu.async_copy(
                    enc_hbm.at[addr_v.at[pl.ds(g * 128, 128)]],
                    gat_v.at[pl.ds(g * 128, 128)], sem)
        return _

    lax.fori_loop(0, NBLK, row_block, 0)

    def gather_drain(g, _):
        pltpu.make_async_copy(
            enc_hbm.at[addr_v.at[pl.ds(g * 128, 128)]],
            gat_v.at[pl.ds(g * 128, 128)], sem).wait()
        return _

    lax.fori_loop(0, 2 * NBLK * 8, gather_drain, 0)

    def scale_group(g, _):
        b2 = (g // 8) % NBLK
        for r8 in range(8):
            s = scale_v[pl.ds(b2 * 128 + r8 * LANES, LANES)]
            v = gat_v[pl.ds(g * 128 + r8 * LANES, LANES)]
            gat_v[pl.ds(g * 128 + r8 * LANES, LANES)] = v * s
        return _

    lax.fori_loop(0, 2 * NBLK * 8, scale_group, 0)

    half_w = NBLK * 8 * 128
    for h in range(2):
        pltpu.sync_copy(
            gat_v.at[pl.ds(h * half_w, half_w)],
            out_hbm.at[pl.ds(h * (BATCH // 128) * 1024 + wid * half_w, half_w)])


@jax.jit
def kernel(image, encoding):
    mesh = plsc.VectorSubcoreMesh(
        core_axis_name="c", subcore_axis_name="s",
        num_cores=NUM_CORES, num_subcores=NUM_SUBCORES)
    k = functools.partial(
        pl.kernel,
        out_type=jax.ShapeDtypeStruct((2 * (BATCH // 128) * 1024,), jnp.float32),
        mesh=mesh,
        scratch_types=[
            pltpu.VMEM((B_PER_W * N_BITS,), jnp.float32),
            pltpu.VMEM((B_PER_W,), jnp.float32),
            pltpu.VMEM((B_PER_W * OUT_DIM,), jnp.int32),
            pltpu.VMEM((B_PER_W * OUT_DIM,), jnp.float32),
            pltpu.SemaphoreType.DMA,
            pltpu.SemaphoreType.DMA,
        ],
        compiler_params=pltpu.CompilerParams(
            needs_layout_passes=False, use_tc_tiling_on_sc=False),
    )(_body)
    enc_flat = (encoding.reshape(8192, 128, 2, 8)
                .transpose(2, 0, 3, 1).reshape(-1))
    out = k(image.T.reshape(-1), enc_flat)
    return (out.reshape(2, BATCH // 128, 8, 128)
            .transpose(1, 3, 0, 2).reshape(BATCH, OUT_DIM))
